# Initial kernel scaffold; baseline (speedup 1.0000x reference)
import jax
import jax.numpy as jnp
from jax.experimental import pallas as pl


def kernel(Z, d, idx_j, emb, in2f_W, fnet_W1, fnet_b1, fnet_W2, fnet_b2, f2out_W1, f2out_b1, f2out_W2, f2out_b2, out_W1, out_W2, out_b2):
    raise NotImplementedError("write your pallas kernel here")


# speedup vs baseline: 4.5797x; 4.5797x over previous
import functools

import jax
import jax.numpy as jnp
from jax import lax
from jax.experimental import pallas as pl
from jax.experimental.pallas import tpu as pltpu
from jax.experimental.pallas import tpu_sc as plsc

RBF_MIN = 0.0
RBF_MAX = 30.0
LOG2 = 0.6931471805599453

NC = 2
NS = 16
NW = NC * NS
SUB = 128


def _ssp(x):
    return jnp.maximum(x, 0.0) + jnp.log1p(jnp.exp(-jnp.abs(x))) - LOG2


def _edge_filters(d, fnet_W1, fnet_b1, fnet_W2, fnet_b2):
    E = d.shape[0]
    L, NRBF, F = fnet_W1.shape
    BLK = 512
    step = (RBF_MAX - RBF_MIN) / (NRBF - 1)
    coeff = -0.5 / step**2

    def body(d_ref, w1_ref, b1_ref, w2_ref, b2_ref, out_ref):
        dcol = d_ref[...]
        offs = (lax.broadcasted_iota(jnp.int32, (1, NRBF), 1).astype(jnp.float32)
                * step + RBF_MIN)
        f = jnp.exp(coeff * (dcol - offs) ** 2)
        for l in range(L):
            u = _ssp(jnp.dot(f, w1_ref[l], preferred_element_type=jnp.float32)
                     + b1_ref[l][None, :])
            out_ref[l] = (jnp.dot(u, w2_ref[l], preferred_element_type=jnp.float32)
                          + b2_ref[l][None, :])

    return pl.pallas_call(
        body,
        grid=(E // BLK,),
        in_specs=[
            pl.BlockSpec((BLK, 1), lambda i: (i, 0)),
            pl.BlockSpec((L, NRBF, F), lambda i: (0, 0, 0)),
            pl.BlockSpec((L, F), lambda i: (0, 0)),
            pl.BlockSpec((L, F, F), lambda i: (0, 0, 0)),
            pl.BlockSpec((L, F), lambda i: (0, 0)),
        ],
        out_specs=pl.BlockSpec((L, BLK, F), lambda i: (0, i, 0)),
        out_shape=jax.ShapeDtypeStruct((L, E, F), jnp.float32),
    )(d.reshape(E, 1), fnet_W1, fnet_b1, fnet_W2, fnet_b2)


def _sc_segment_sum(wf, idx1, zeros, N):
    L, E, F = wf.shape
    NSUB = E // SUB
    T0 = NSUB // NW
    REM = NSUB - T0 * NW
    RLO = (N // NS) // 8 * 8
    NHI = (N - RLO * NS) // 8

    mesh = plsc.VectorSubcoreMesh(core_axis_name="c", subcore_axis_name="s")

    @functools.partial(
        pl.kernel,
        out_type=jax.ShapeDtypeStruct((L, NC, N, F), jnp.float32),
        mesh=mesh,
        scratch_types=[
            pltpu.VMEM((SUB,), jnp.int32),
            pltpu.VMEM((SUB,), jnp.int32),
            pltpu.VMEM((SUB, F), jnp.float32),
            pltpu.VMEM((SUB, F), jnp.float32),
            pltpu.VMEM_SHARED((N, F), jnp.float32),
            pltpu.SemaphoreType.DMA,
            pltpu.SemaphoreType.DMA,
        ],
    )
    def seg(wf_hbm, idx_hbm, z_hbm, out_hbm,
            idxb0, idxb1, rows0, rows1, acc, sem0, sem1):
        c = lax.axis_index("c")
        s = lax.axis_index("s")
        w = s * NC + c
        tw = T0 + jnp.where(w < REM, 1, 0)
        idx_bufs = (idxb0, idxb1)
        row_bufs = (rows0, rows1)
        sems = (sem0, sem1)
        row0 = jnp.where(s < NHI, s * (RLO + 8), NHI * 8 + s * RLO)

        for l in range(L):
            @pl.when(s < NHI)
            def _():
                pltpu.sync_copy(z_hbm, acc.at[pl.ds(row0, RLO + 8)])

            @pl.when(s >= NHI)
            def _():
                pltpu.sync_copy(z_hbm.at[pl.ds(0, RLO)], acc.at[pl.ds(row0, RLO)])

            plsc.subcore_barrier()

            def start(b, t):
                r = w + NW * t
                pltpu.async_copy(idx_hbm.at[pl.ds(r * SUB, SUB)], idx_bufs[b],
                                 sems[b])
                pltpu.async_copy(wf_hbm.at[l].at[pl.ds(r * SUB, SUB)],
                                 row_bufs[b], sems[b])

            def drain(b):
                pltpu.make_async_copy(idx_hbm.at[pl.ds(0, SUB)], idx_bufs[b],
                                      sems[b]).wait()
                pltpu.make_async_copy(wf_hbm.at[l].at[pl.ds(0, SUB)],
                                      row_bufs[b], sems[b]).wait()

            start(0, 0)
            start(1, 1)

            def gbody(g, carry):
                for b in range(2):
                    t = 2 * g + b
                    drain(b)
                    pltpu.sync_copy(row_bufs[b], acc.at[idx_bufs[b]], add=True)
                    nxt = t + 2

                    @pl.when(nxt < tw)
                    def _():
                        start(b, nxt)
                return carry

            lax.fori_loop(0, T0 // 2, gbody, 0)

            @pl.when(w < REM)
            def _():
                drain(0)
                pltpu.sync_copy(row_bufs[0], acc.at[idx_bufs[0]], add=True)

            plsc.subcore_barrier()

            @pl.when(s < NHI)
            def _():
                pltpu.sync_copy(acc.at[pl.ds(row0, RLO + 8)],
                                out_hbm.at[l].at[c].at[pl.ds(row0, RLO + 8)])

            @pl.when(s >= NHI)
            def _():
                pltpu.sync_copy(acc.at[pl.ds(row0, RLO)],
                                out_hbm.at[l].at[c].at[pl.ds(row0, RLO)])

            plsc.subcore_barrier()

    return seg(wf, idx1, zeros)


def _node_net(Z2, emb_p, in2f_W, parts, f2out_W1, f2out_b1, f2out_W2,
              f2out_b2, out_W1, out_W2, out_b2):
    L, NCp, N, F = parts.shape
    MZ = emb_p.shape[0]
    H = out_W1.shape[1]
    BLK = 1000

    def body(z_ref, emb_ref, in2f_ref, parts_ref, w1_ref, b1_ref, w2_ref,
             b2_ref, ow1_ref, ow2_ref, ob2_ref, out_ref):
        z = z_ref[...]
        ids = lax.broadcasted_iota(jnp.int32, (1, MZ), 1)
        oh = (z == ids).astype(jnp.float32)
        x = jnp.dot(oh, emb_ref[...], preferred_element_type=jnp.float32)
        for l in range(L):
            h = jnp.dot(x, in2f_ref[l], preferred_element_type=jnp.float32)
            agg = h * (parts_ref[l, 0] + parts_ref[l, 1])
            t = _ssp(jnp.dot(agg, w1_ref[l], preferred_element_type=jnp.float32)
                     + b1_ref[l][None, :])
            x = x + jnp.dot(t, w2_ref[l], preferred_element_type=jnp.float32) \
                + b2_ref[l][None, :]
        y = _ssp(jnp.dot(x, ow1_ref[...], preferred_element_type=jnp.float32))
        out_ref[...] = (jnp.dot(y, ow2_ref[...], preferred_element_type=jnp.float32)
                        + ob2_ref[0, 0])

    return pl.pallas_call(
        body,
        grid=(Z2.shape[0] // BLK,),
        in_specs=[
            pl.BlockSpec((BLK, 1), lambda i: (i, 0)),
            pl.BlockSpec((MZ, F), lambda i: (0, 0)),
            pl.BlockSpec((L, F, F), lambda i: (0, 0, 0)),
            pl.BlockSpec((L, NCp, BLK, F), lambda i: (0, 0, i, 0)),
            pl.BlockSpec((L, F, F), lambda i: (0, 0, 0)),
            pl.BlockSpec((L, F), lambda i: (0, 0)),
            pl.BlockSpec((L, F, F), lambda i: (0, 0, 0)),
            pl.BlockSpec((L, F), lambda i: (0, 0)),
            pl.BlockSpec((F, H), lambda i: (0, 0)),
            pl.BlockSpec((H, 1), lambda i: (0, 0)),
            pl.BlockSpec((1, 1), lambda i: (0, 0)),
        ],
        out_specs=pl.BlockSpec((BLK, 1), lambda i: (i, 0)),
        out_shape=jax.ShapeDtypeStruct((N, 1), jnp.float32),
    )(Z2, emb_p, in2f_W, parts, f2out_W1, f2out_b1, f2out_W2, f2out_b2,
      out_W1, out_W2, out_b2.reshape(1, 1))


def kernel(Z, d, idx_j, emb, in2f_W, fnet_W1, fnet_b1, fnet_W2, fnet_b2,
           f2out_W1, f2out_b1, f2out_W2, f2out_b2, out_W1, out_W2, out_b2):
    N = Z.shape[0]
    E = d.shape[0]
    F = emb.shape[1]

    wf = _edge_filters(d, fnet_W1, fnet_b1, fnet_W2, fnet_b2)

    idx1 = idx_j.astype(jnp.int32)
    zeros = jnp.zeros(((N // NS) // 8 * 8 + 8, F), jnp.float32)
    parts = _sc_segment_sum(wf, idx1, zeros, N)

    emb_p = jnp.zeros((128, F), jnp.float32).at[:emb.shape[0]].set(emb)
    out = _node_net(Z.astype(jnp.int32).reshape(N, 1), emb_p, in2f_W, parts,
                    f2out_W1, f2out_b1, f2out_W2, f2out_b2,
                    out_W1, out_W2, out_b2)
    return out.reshape(N)
